# baseline (device time: 76116 ns/iter reference)
import jax
import jax.numpy as jnp
from jax import lax
from jax.experimental import pallas as pl
from jax.experimental.pallas import tpu as pltpu


def _exchange_body(x_ref, d_ref, peer_x_ref, peer_d_ref, send_sems, recv_sems):
    my_xi = lax.axis_index("x")
    my_yi = lax.axis_index("y")
    peer = (1 - my_xi, my_yi)

    barrier = pltpu.get_barrier_semaphore()
    pl.semaphore_signal(
        barrier, inc=1, device_id=peer, device_id_type=pl.DeviceIdType.MESH
    )
    pl.semaphore_wait(barrier, 1)

    r1 = pltpu.make_async_remote_copy(
        src_ref=x_ref,
        dst_ref=peer_x_ref,
        send_sem=send_sems.at[0],
        recv_sem=recv_sems.at[0],
        device_id=peer,
        device_id_type=pl.DeviceIdType.MESH,
    )
    r1.start()
    r2 = pltpu.make_async_remote_copy(
        src_ref=d_ref,
        dst_ref=peer_d_ref,
        send_sem=send_sems.at[1],
        recv_sem=recv_sems.at[1],
        device_id=peer,
        device_id_type=pl.DeviceIdType.MESH,
    )
    r2.start()
    r1.wait()
    r2.wait()


def kernel(x, dest):
    m, n = x.shape
    xb = x.astype(jnp.bfloat16)
    d2 = dest.reshape(16, 128)

    peer_x, peer_d2 = pl.pallas_call(
        _exchange_body,
        out_shape=(
            jax.ShapeDtypeStruct((m, n), jnp.bfloat16),
            jax.ShapeDtypeStruct((16, 128), jnp.int32),
        ),
        in_specs=[
            pl.BlockSpec(memory_space=pltpu.VMEM),
            pl.BlockSpec(memory_space=pltpu.VMEM),
        ],
        out_specs=(
            pl.BlockSpec(memory_space=pltpu.VMEM),
            pl.BlockSpec(memory_space=pltpu.VMEM),
        ),
        scratch_shapes=[
            pltpu.SemaphoreType.DMA((2,)),
            pltpu.SemaphoreType.DMA((2,)),
        ],
        compiler_params=pltpu.CompilerParams(collective_id=0),
    )(xb, d2)

    peer_dest = peer_d2.reshape(-1)
    my_xi = lax.axis_index("x")
    is0 = my_xi == 0

    full_x = jnp.concatenate(
        [jnp.where(is0, xb, peer_x), jnp.where(is0, peer_x, xb)], axis=0
    )
    full_dest = jnp.concatenate(
        [jnp.where(is0, dest, peer_dest), jnp.where(is0, peer_dest, dest)]
    )
    order = jnp.argsort(full_dest, stable=True)
    my_rows = lax.dynamic_slice_in_dim(order, my_xi * m, m)
    return jnp.take(full_x, my_rows, axis=0).astype(jnp.float32)


# device time: 62866 ns/iter; 1.2108x vs baseline; 1.2108x over previous
import jax
import jax.numpy as jnp
from jax import lax
from jax.experimental import pallas as pl
from jax.experimental.pallas import tpu as pltpu

NBITS = 9


def _body(scal_ref, xs_ref, recv_ref, send_sems, recv_sems):
    my_xi = lax.axis_index("x")
    my_yi = lax.axis_index("y")
    peer = (1 - my_xi, my_yi)

    k8 = scal_ref[0]

    barrier = pltpu.get_barrier_semaphore()
    pl.semaphore_signal(
        barrier, inc=1, device_id=peer, device_id_type=pl.DeviceIdType.MESH
    )
    pl.semaphore_wait(barrier, 1)

    def rdma_for_bit(b):
        rows = 8 << b
        off = ((k8 >> (b + 1)) << (b + 1)) * 8
        return pltpu.make_async_remote_copy(
            src_ref=xs_ref.at[pl.ds(off, rows), :],
            dst_ref=recv_ref.at[pl.ds(off, rows), :],
            send_sem=send_sems.at[b],
            recv_sem=recv_sems.at[b],
            device_id=peer,
            device_id_type=pl.DeviceIdType.MESH,
        )

    for b in range(NBITS - 1, -1, -1):
        @pl.when(((k8 >> b) & 1) == 1)
        def _(b=b):
            rdma_for_bit(b).start()

    for b in range(NBITS - 1, -1, -1):
        @pl.when(((k8 >> b) & 1) == 1)
        def _(b=b):
            rdma_for_bit(b).wait()


def kernel(x, dest):
    m, n = x.shape
    my_xi = lax.axis_index("x")

    to_peer = dest != my_xi
    k = jnp.sum(to_peer.astype(jnp.int32))
    perm = jnp.argsort(jnp.logical_not(to_peer), stable=True)
    xs = x.astype(jnp.bfloat16)[perm]

    k8 = (k + 7) >> 3
    scal = k8.astype(jnp.int32).reshape(1)

    recv = pl.pallas_call(
        _body,
        out_shape=jax.ShapeDtypeStruct((m, n), jnp.bfloat16),
        in_specs=[
            pl.BlockSpec(memory_space=pltpu.SMEM),
            pl.BlockSpec(memory_space=pltpu.VMEM),
        ],
        out_specs=pl.BlockSpec(memory_space=pltpu.VMEM),
        scratch_shapes=[
            pltpu.SemaphoreType.DMA((NBITS,)),
            pltpu.SemaphoreType.DMA((NBITS,)),
        ],
        compiler_params=pltpu.CompilerParams(collective_id=0),
    )(scal, xs)

    j = jnp.arange(m)
    n_keep = m - k
    idx0 = jnp.where(j < n_keep, k + j, m + j - n_keep)
    idx1 = jnp.where(j < k, m + j, j)
    idx = jnp.where(my_xi == 0, idx0, idx1)
    return jnp.take(jnp.concatenate([xs, recv], axis=0), idx, axis=0)
